# baseline (speedup 1.0000x reference)
import functools

import jax
import jax.numpy as jnp
from jax import lax
from jax.experimental import pallas as pl
from jax.experimental.pallas import tpu as pltpu
from jax.experimental.pallas import tpu_sc as plsc

_NC = 2
_NS = 16
_NW = _NC * _NS
_CH = 128


def _sc_gather(table128, idx2):
    nrow, ch = idx2.shape
    assert ch == _CH
    BL = nrow * _CH
    V, DP = table128.shape
    per_w = BL // _NW
    nch = per_w // _CH
    assert per_w % _CH == 0 and nch >= 3
    npairs = (nch - 1) // 2
    mesh = plsc.VectorSubcoreMesh(core_axis_name="c", subcore_axis_name="s")

    @functools.partial(
        pl.kernel,
        mesh=mesh,
        compiler_params=pltpu.CompilerParams(use_tc_tiling_on_sc=False),
        out_type=jax.ShapeDtypeStruct((BL, DP), jnp.float32),
        scratch_types=[
            pltpu.VMEM((nch, _CH), jnp.int32),
            pltpu.VMEM((_CH, DP), jnp.float32),
            pltpu.VMEM((_CH, DP), jnp.float32),
            pltpu.SemaphoreType.DMA,
            pltpu.SemaphoreType.DMA,
        ],
    )
    def gather_kernel(table_hbm, idx_hbm, out_hbm, idx_v, buf0, buf1, sem0, sem1):
        wid = lax.axis_index("s") * _NC + lax.axis_index("c")
        base = wid * per_w
        pltpu.sync_copy(idx_hbm.at[pl.ds(wid * nch, nch)], idx_v)
        bufs = (buf0, buf1)
        sems = (sem0, sem1)

        def gather_start(c, b):
            pltpu.make_async_copy(
                table_hbm.at[idx_v.at[c]], bufs[b], sems[b]).start()

        def gather_wait_and_flush(c, b):
            pltpu.make_async_copy(
                table_hbm.at[idx_v.at[c]], bufs[b], sems[b]).wait()
            pltpu.sync_copy(bufs[b], out_hbm.at[pl.ds(base + c * _CH, _CH)])

        gather_start(0, 0)
        gather_start(1, 1)

        def body(g, carry):
            c = g * 2
            for b in range(2):
                gather_wait_and_flush(c + b, b)

                @pl.when(c + b + 2 < nch)
                def _():
                    gather_start(c + b + 2, b)
            return carry

        lax.fori_loop(0, npairs, body, 0)
        for c in range(2 * npairs, nch):
            gather_wait_and_flush(c, c % 2)

    return gather_kernel(table128, idx2)


def _mlp_body(g_ref, r_ref, r2e_ref, w1t_ref, b1_ref, w2t_ref, b2_ref,
              out_ref, a_ref, bm_ref, of_ref, ms_ref, *, bb, L, D, NR):
    rows = bb * L
    nr = rows // 128

    @pl.when(pl.program_id(0) == 0)
    def _init():
        ja = lax.broadcasted_iota(jnp.int32, (rows, nr), 0) >> 7
        a_ref[...] = (ja == lax.broadcasted_iota(
            jnp.int32, (rows, nr), 1)).astype(jnp.bfloat16)
        jb = lax.broadcasted_iota(jnp.int32, (rows, 128), 0) & 127
        bm_ref[...] = (jb == lax.broadcasted_iota(
            jnp.int32, (rows, 128), 1)).astype(jnp.bfloat16)
        of_ref[...] = lax.broadcasted_iota(
            jnp.int32, (rows, 8), 1).astype(jnp.float32)
        ju = lax.broadcasted_iota(jnp.int32, (bb, rows), 1) // L
        ms_ref[...] = (ju == lax.broadcasted_iota(
            jnp.int32, (bb, rows), 0)).astype(jnp.bfloat16)

    radj = r_ref[0] - 1
    radj = jnp.where(radj < 0, radj + NR, radj).astype(jnp.bfloat16)
    Y1 = jnp.dot(a_ref[...], radj, preferred_element_type=jnp.float32)
    y8 = jnp.dot((Y1.astype(jnp.bfloat16) * bm_ref[...]),
                 jnp.ones((128, 8), jnp.bfloat16),
                 preferred_element_type=jnp.float32)
    oh = (y8 == of_ref[...]).astype(jnp.bfloat16)
    g = g_ref[...][:, :D].astype(jnp.bfloat16)
    w1t = w1t_ref[...]
    tr = jnp.dot(r2e_ref[...], w1t[D:, :],
                 preferred_element_type=jnp.float32)
    tr = (tr + b1_ref[...]).astype(jnp.bfloat16)
    h = jnp.dot(g, w1t[:D, :].astype(jnp.bfloat16),
                preferred_element_type=jnp.float32)
    h = h + jnp.dot(oh, tr, preferred_element_type=jnp.float32)
    h = jnp.maximum(h, 0.0).astype(jnp.bfloat16)
    h = jnp.dot(h, w2t_ref[...].astype(jnp.bfloat16),
                preferred_element_type=jnp.float32)
    h = jnp.maximum(h + b2_ref[...], 0.0).astype(jnp.bfloat16)
    out_ref[...] = jnp.dot(ms_ref[...], h,
                           preferred_element_type=jnp.float32) * (1.0 / L)


def _mlp_call(g, r3, r2e_pad, w1t, b1r, w2t, b2r, *, bb, L, D, NR):
    Bc = r3.shape[0] * bb
    rows = bb * L
    return pl.pallas_call(
        functools.partial(_mlp_body, bb=bb, L=L, D=D, NR=NR),
        grid=(Bc // bb,),
        in_specs=[
            pl.BlockSpec((rows, 128), lambda i: (i, 0)),
            pl.BlockSpec((1, rows // 128, 128), lambda i: (i, 0, 0)),
            pl.BlockSpec((8, D), lambda i: (0, 0)),
            pl.BlockSpec((2 * D, D), lambda i: (0, 0)),
            pl.BlockSpec((1, D), lambda i: (0, 0)),
            pl.BlockSpec((D, D), lambda i: (0, 0)),
            pl.BlockSpec((1, D), lambda i: (0, 0)),
        ],
        out_specs=pl.BlockSpec((bb, D), lambda i: (i, 0)),
        out_shape=jax.ShapeDtypeStruct((Bc, D), jnp.float32),
        scratch_shapes=[
            pltpu.VMEM((rows, rows // 128), jnp.bfloat16),
            pltpu.VMEM((rows, 128), jnp.bfloat16),
            pltpu.VMEM((rows, 8), jnp.float32),
            pltpu.VMEM((bb, rows), jnp.bfloat16),
        ],
    )(g, r3, r2e_pad, w1t, b1r, w2t, b2r)


def kernel(history_uv, history_r, v2e_w, r2e_w, W1, b1, W2, b2):
    B, L = history_uv.shape
    V, D = v2e_w.shape
    NR = r2e_w.shape[0]
    BL = B * L
    bb = 128
    rows = bb * L
    NCHUNK = 2

    table128 = jnp.pad(v2e_w, ((0, 0), (0, 128 - D)))
    idx3 = history_uv.reshape(NCHUNK, BL // NCHUNK // 128, 128).astype(jnp.int32)
    r4 = history_r.reshape(
        NCHUNK, B // bb // NCHUNK, rows // 128, 128).astype(jnp.int32)
    r2e_pad = jnp.pad(r2e_w, ((0, 8 - NR), (0, 0)))
    w1t = W1.T
    w2t = W2.T
    b1r = b1.reshape(1, D)
    b2r = b2.reshape(1, D)

    outs = []
    for k in range(NCHUNK):
        g_k = _sc_gather(table128, idx3[k])
        outs.append(_mlp_call(g_k, r4[k], r2e_pad, w1t, b1r, w2t, b2r,
                              bb=bb, L=L, D=D, NR=NR))
    return jnp.concatenate(outs, axis=0)


# speedup vs baseline: 1.1306x; 1.1306x over previous
import functools

import jax
import jax.numpy as jnp
from jax import lax
from jax.experimental import pallas as pl
from jax.experimental.pallas import tpu as pltpu
from jax.experimental.pallas import tpu_sc as plsc

_NC = 2
_NS = 16
_NW = _NC * _NS
_CH = 128


def _sc_gather(table128, idx2):
    nrow, ch = idx2.shape
    assert ch == _CH
    BL = nrow * _CH
    V, DP = table128.shape
    per_w = BL // _NW
    nch = per_w // _CH
    assert per_w % _CH == 0 and nch >= 3
    npairs = (nch - 1) // 2
    mesh = plsc.VectorSubcoreMesh(core_axis_name="c", subcore_axis_name="s")

    @functools.partial(
        pl.kernel,
        mesh=mesh,
        compiler_params=pltpu.CompilerParams(use_tc_tiling_on_sc=False),
        out_type=jax.ShapeDtypeStruct((BL, DP), jnp.float32),
        scratch_types=[
            pltpu.VMEM((nch, _CH), jnp.int32),
            pltpu.VMEM((_CH, DP), jnp.float32),
            pltpu.VMEM((_CH, DP), jnp.float32),
            pltpu.SemaphoreType.DMA,
            pltpu.SemaphoreType.DMA,
        ],
    )
    def gather_kernel(table_hbm, idx_hbm, out_hbm, idx_v, buf0, buf1, sem0, sem1):
        wid = lax.axis_index("s") * _NC + lax.axis_index("c")
        base = wid * per_w
        pltpu.sync_copy(idx_hbm.at[pl.ds(wid * nch, nch)], idx_v)
        bufs = (buf0, buf1)
        sems = (sem0, sem1)

        def gather_start(c, b):
            pltpu.make_async_copy(
                table_hbm.at[idx_v.at[c]], bufs[b], sems[b]).start()

        def gather_wait_and_flush(c, b):
            pltpu.make_async_copy(
                table_hbm.at[idx_v.at[c]], bufs[b], sems[b]).wait()
            pltpu.sync_copy(bufs[b], out_hbm.at[pl.ds(base + c * _CH, _CH)])

        gather_start(0, 0)
        gather_start(1, 1)

        def body(g, carry):
            c = g * 2
            for b in range(2):
                gather_wait_and_flush(c + b, b)

                @pl.when(c + b + 2 < nch)
                def _():
                    gather_start(c + b + 2, b)
            return carry

        lax.fori_loop(0, npairs, body, 0)
        for c in range(2 * npairs, nch):
            gather_wait_and_flush(c, c % 2)

    return gather_kernel(table128, idx2)


def _mlp_body(g_ref, r_ref, r2e_ref, w1t_ref, b1_ref, w2t_ref, b2_ref,
              out_ref, a_ref, bm_ref, of_ref, ms_ref, *, bb, L, D, NR):
    rows = bb * L
    nr = rows // 128

    @pl.when(pl.program_id(0) == 0)
    def _init():
        ja = lax.broadcasted_iota(jnp.int32, (rows, nr), 0) >> 7
        a_ref[...] = (ja == lax.broadcasted_iota(
            jnp.int32, (rows, nr), 1)).astype(jnp.bfloat16)
        jb = lax.broadcasted_iota(jnp.int32, (rows, 128), 0) & 127
        bm_ref[...] = (jb == lax.broadcasted_iota(
            jnp.int32, (rows, 128), 1)).astype(jnp.float32)
        of_ref[...] = lax.broadcasted_iota(
            jnp.int32, (rows, 8), 1).astype(jnp.float32)
        ju = lax.broadcasted_iota(jnp.int32, (bb, rows), 1) // L
        ms_ref[...] = (ju == lax.broadcasted_iota(
            jnp.int32, (bb, rows), 0)).astype(jnp.bfloat16)

    radj = r_ref[0] - 1
    radj = jnp.where(radj < 0, radj + NR, radj).astype(jnp.bfloat16)
    Y1 = jnp.dot(a_ref[...], radj, preferred_element_type=jnp.float32)
    y = jnp.sum(Y1 * bm_ref[...], axis=1, keepdims=True)
    oh = (y == of_ref[...]).astype(jnp.float32)
    g = g_ref[...][:, :D]
    w1t = w1t_ref[...]
    tr = jnp.dot(r2e_ref[...], w1t[D:, :],
                 preferred_element_type=jnp.float32)
    tr = tr + b1_ref[...]
    h = jnp.dot(g, w1t[:D, :], preferred_element_type=jnp.float32)
    h = h + jnp.dot(oh, tr, preferred_element_type=jnp.float32)
    h = jnp.maximum(h, 0.0)
    h = jnp.dot(h, w2t_ref[...], preferred_element_type=jnp.float32)
    h = jnp.maximum(h + b2_ref[...], 0.0)
    out_ref[...] = jnp.dot(ms_ref[...], h.astype(jnp.bfloat16),
                           preferred_element_type=jnp.float32) * (1.0 / L)


def _mlp_call(g, r3, r2e_pad, w1t, b1r, w2t, b2r, *, bb, L, D, NR):
    Bc = r3.shape[0] * bb
    rows = bb * L
    return pl.pallas_call(
        functools.partial(_mlp_body, bb=bb, L=L, D=D, NR=NR),
        grid=(Bc // bb,),
        in_specs=[
            pl.BlockSpec((rows, 128), lambda i: (i, 0)),
            pl.BlockSpec((1, rows // 128, 128), lambda i: (i, 0, 0)),
            pl.BlockSpec((8, D), lambda i: (0, 0)),
            pl.BlockSpec((2 * D, D), lambda i: (0, 0)),
            pl.BlockSpec((1, D), lambda i: (0, 0)),
            pl.BlockSpec((D, D), lambda i: (0, 0)),
            pl.BlockSpec((1, D), lambda i: (0, 0)),
        ],
        out_specs=pl.BlockSpec((bb, D), lambda i: (i, 0)),
        out_shape=jax.ShapeDtypeStruct((Bc, D), jnp.float32),
        scratch_shapes=[
            pltpu.VMEM((rows, rows // 128), jnp.bfloat16),
            pltpu.VMEM((rows, 128), jnp.float32),
            pltpu.VMEM((rows, 8), jnp.float32),
            pltpu.VMEM((bb, rows), jnp.bfloat16),
        ],
    )(g, r3, r2e_pad, w1t, b1r, w2t, b2r)


def kernel(history_uv, history_r, v2e_w, r2e_w, W1, b1, W2, b2):
    B, L = history_uv.shape
    V, D = v2e_w.shape
    NR = r2e_w.shape[0]
    BL = B * L
    bb = 128
    rows = bb * L
    NCHUNK = 2

    table128 = jnp.pad(v2e_w, ((0, 0), (0, 128 - D)))
    idx3 = history_uv.reshape(NCHUNK, BL // NCHUNK // 128, 128).astype(jnp.int32)
    r4 = history_r.reshape(
        NCHUNK, B // bb // NCHUNK, rows // 128, 128).astype(jnp.int32)
    r2e_pad = jnp.pad(r2e_w, ((0, 8 - NR), (0, 0)))
    w1t = W1.T
    w2t = W2.T
    b1r = b1.reshape(1, D)
    b2r = b2.reshape(1, D)

    outs = []
    for k in range(NCHUNK):
        g_k = _sc_gather(table128, idx3[k])
        outs.append(_mlp_call(g_k, r4[k], r2e_pad, w1t, b1r, w2t, b2r,
                              bb=bb, L=L, D=D, NR=NR))
    return jnp.concatenate(outs, axis=0)
